# Initial kernel scaffold; baseline (speedup 1.0000x reference)
import jax
import jax.numpy as jnp
from jax.experimental import pallas as pl


def kernel(h, edge_index, edge_weight, W, b, gamma, beta):
    raise NotImplementedError("write your pallas kernel here")


# speedup vs baseline: 2.6662x; 2.6662x over previous
import functools

import jax
import jax.numpy as jnp
from jax import lax
from jax.experimental import pallas as pl
from jax.experimental.pallas import tpu as pltpu
from jax.experimental.pallas import tpu_sc as plsc

N_NODES = 10000
N_EDGES = 320000
DIM = 128
BN_EPS = 1e-5

NC = 2
NS = 16
NW = NC * NS
E_PAD = 327680
EPW = E_PAD // NW
CH = 128
NCH = EPW // CH
N_PAD = 10112
RPT = N_PAD // NS

ROW_BLK = 2000
N_BLKS = N_NODES // ROW_BLK


def _mm_body(h_ref, w_ref, b_ref, o_ref):
    o_ref[...] = (
        jnp.dot(h_ref[...], w_ref[...], preferred_element_type=jnp.float32)
        + b_ref[...]
    )


def _project(h, W, b2):
    return pl.pallas_call(
        _mm_body,
        grid=(N_BLKS,),
        in_specs=[
            pl.BlockSpec((ROW_BLK, DIM), lambda i: (i, 0)),
            pl.BlockSpec((DIM, DIM), lambda i: (0, 0)),
            pl.BlockSpec((1, DIM), lambda i: (0, 0)),
        ],
        out_specs=pl.BlockSpec((ROW_BLK, DIM), lambda i: (i, 0)),
        out_shape=jax.ShapeDtypeStruct((N_NODES, DIM), jnp.float32),
    )(h, W, b2)


def _sc_aggregate(h2, src, dst, ew):
    mesh = plsc.VectorSubcoreMesh(
        core_axis_name="c", subcore_axis_name="s", num_cores=NC, num_subcores=NS
    )

    @functools.partial(
        pl.kernel,
        out_type=jax.ShapeDtypeStruct((NC, N_PAD, DIM), jnp.float32),
        mesh=mesh,
        scratch_types=[
            pltpu.VMEM((CH,), jnp.int32),
            pltpu.VMEM((CH,), jnp.int32),
            pltpu.VMEM((CH,), jnp.float32),
            pltpu.VMEM((CH, DIM), jnp.float32),
            pltpu.VMEM((8, DIM), jnp.float32),
            pltpu.VMEM_SHARED((N_PAD, DIM), jnp.float32),
            pltpu.SemaphoreType.DMA,
        ],
        compiler_params=pltpu.CompilerParams(needs_layout_passes=False),
    )
    def body(h2_hbm, src_hbm, dst_hbm, ew_hbm, out_hbm,
             src_v, dst_v, ew_v, rows_v, zbuf_v, agg_sh, sem):
        c = lax.axis_index("c")
        s = lax.axis_index("s")
        wid = s * NC + c

        for r in range(8):
            for j in range(DIM // 16):
                zbuf_v[r, pl.ds(j * 16, 16)] = jnp.zeros((16,), jnp.float32)

        def zcopy(k, _):
            pltpu.sync_copy(zbuf_v, agg_sh.at[pl.ds(s * RPT + k * 8, 8)])
            return 0
        lax.fori_loop(0, RPT // 8, zcopy, 0)
        plsc.subcore_barrier()

        base = wid * EPW

        def chunk(kk, _):
            off = base + kk * CH
            pltpu.sync_copy(src_hbm.at[pl.ds(off, CH)], src_v)
            pltpu.sync_copy(dst_hbm.at[pl.ds(off, CH)], dst_v)
            pltpu.sync_copy(ew_hbm.at[pl.ds(off, CH)], ew_v)
            pltpu.async_copy(h2_hbm.at[src_v], rows_v, sem).wait()

            def rowm(i, _):
                wv = plsc.load_gather(ew_v, [lax.broadcast(i, (16,))])
                for j in range(DIM // 16):
                    sl = pl.ds(j * 16, 16)
                    rows_v[i, sl] = rows_v[i, sl] * wv
                return 0
            lax.fori_loop(0, CH, rowm, 0)

            pltpu.sync_copy(rows_v, agg_sh.at[dst_v], add=True)
            return 0
        lax.fori_loop(0, NCH, chunk, 0)

        plsc.subcore_barrier()
        pltpu.sync_copy(
            agg_sh.at[pl.ds(s * RPT, RPT)],
            out_hbm.at[c, pl.ds(s * RPT, RPT)],
        )

    return body(h2, src, dst, ew)


def _comb_body(h2_ref, p0_ref, p1_ref, h3_ref, sum_ref, sq_ref):
    i = pl.program_id(0)
    x = h2_ref[...] + p0_ref[...] + p1_ref[...]
    h3_ref[...] = x

    @pl.when(i == 0)
    def _():
        sum_ref[...] = jnp.zeros_like(sum_ref)
        sq_ref[...] = jnp.zeros_like(sq_ref)

    sum_ref[0:1, :] += jnp.sum(x, axis=0, keepdims=True)
    sq_ref[0:1, :] += jnp.sum(x * x, axis=0, keepdims=True)


def _combine(h2, p0, p1):
    return pl.pallas_call(
        _comb_body,
        grid=(N_BLKS,),
        in_specs=[
            pl.BlockSpec((ROW_BLK, DIM), lambda i: (i, 0)),
            pl.BlockSpec((ROW_BLK, DIM), lambda i: (i, 0)),
            pl.BlockSpec((ROW_BLK, DIM), lambda i: (i, 0)),
        ],
        out_specs=[
            pl.BlockSpec((ROW_BLK, DIM), lambda i: (i, 0)),
            pl.BlockSpec((8, DIM), lambda i: (0, 0)),
            pl.BlockSpec((8, DIM), lambda i: (0, 0)),
        ],
        out_shape=[
            jax.ShapeDtypeStruct((N_NODES, DIM), jnp.float32),
            jax.ShapeDtypeStruct((8, DIM), jnp.float32),
            jax.ShapeDtypeStruct((8, DIM), jnp.float32),
        ],
    )(h2, p0, p1)


def _bn_body(h3_ref, sum_ref, sq_ref, g_ref, be_ref, o_ref):
    n = jnp.float32(N_NODES)
    mean = sum_ref[0:1, :] / n
    var = sq_ref[0:1, :] / n - mean * mean
    inv = lax.rsqrt(var + BN_EPS)
    o_ref[...] = g_ref[...] * (h3_ref[...] - mean) * inv + be_ref[...]


def _bn_apply(h3, ssum, ssq, g2, be2):
    return pl.pallas_call(
        _bn_body,
        grid=(N_BLKS,),
        in_specs=[
            pl.BlockSpec((ROW_BLK, DIM), lambda i: (i, 0)),
            pl.BlockSpec((8, DIM), lambda i: (0, 0)),
            pl.BlockSpec((8, DIM), lambda i: (0, 0)),
            pl.BlockSpec((1, DIM), lambda i: (0, 0)),
            pl.BlockSpec((1, DIM), lambda i: (0, 0)),
        ],
        out_specs=pl.BlockSpec((ROW_BLK, DIM), lambda i: (i, 0)),
        out_shape=jax.ShapeDtypeStruct((N_NODES, DIM), jnp.float32),
    )(h3, ssum, ssq, g2, be2)


def kernel(h, edge_index, edge_weight, W, b, gamma, beta):
    src = edge_index[0].astype(jnp.int32)
    dst = edge_index[1].astype(jnp.int32)
    pad = E_PAD - N_EDGES
    src = jnp.concatenate([src, jnp.zeros((pad,), jnp.int32)])
    dst = jnp.concatenate([dst, jnp.zeros((pad,), jnp.int32)])
    ew = jnp.concatenate([edge_weight.astype(jnp.float32),
                          jnp.zeros((pad,), jnp.float32)])

    h2 = _project(h, W, b.reshape(1, DIM))
    parts = _sc_aggregate(h2, src, dst, ew)
    h3, ssum, ssq = _combine(h2, parts[0, :N_NODES], parts[1, :N_NODES])
    return _bn_apply(h3, ssum, ssq, gamma.reshape(1, DIM),
                     beta.reshape(1, DIM))


# speedup vs baseline: 3.2057x; 1.2023x over previous
import functools

import jax
import jax.numpy as jnp
from jax import lax
from jax.experimental import pallas as pl
from jax.experimental.pallas import tpu as pltpu
from jax.experimental.pallas import tpu_sc as plsc

N_NODES = 10000
N_EDGES = 320000
DIM = 128
BN_EPS = 1e-5

NC = 2
NS = 16
NW = NC * NS
E_PAD = 327680
EPW = E_PAD // NW
CH = 128
NCH = EPW // CH
N_PAD = 10112
RPT = N_PAD // NS

ROW_BLK = 2000
N_BLKS = N_NODES // ROW_BLK


def _mm_body(h_ref, w_ref, b_ref, o_ref):
    o_ref[...] = (
        jnp.dot(h_ref[...], w_ref[...], preferred_element_type=jnp.float32)
        + b_ref[...]
    )


def _project(h, W, b2):
    return pl.pallas_call(
        _mm_body,
        grid=(N_BLKS,),
        in_specs=[
            pl.BlockSpec((ROW_BLK, DIM), lambda i: (i, 0)),
            pl.BlockSpec((DIM, DIM), lambda i: (0, 0)),
            pl.BlockSpec((1, DIM), lambda i: (0, 0)),
        ],
        out_specs=pl.BlockSpec((ROW_BLK, DIM), lambda i: (i, 0)),
        out_shape=jax.ShapeDtypeStruct((N_NODES, DIM), jnp.float32),
    )(h, W, b2)


def _sc_aggregate(h2, edata):
    mesh = plsc.VectorSubcoreMesh(
        core_axis_name="c", subcore_axis_name="s", num_cores=NC, num_subcores=NS
    )

    @functools.partial(
        pl.kernel,
        out_type=jax.ShapeDtypeStruct((NC, N_PAD, DIM), jnp.float32),
        mesh=mesh,
        scratch_types=[
            pltpu.VMEM((3, CH), jnp.int32),
            pltpu.VMEM((3, CH), jnp.int32),
            pltpu.VMEM((CH, DIM), jnp.float32),
            pltpu.VMEM((CH, DIM), jnp.float32),
            pltpu.VMEM((CH,), jnp.int32),
            pltpu.VMEM((CH,), jnp.int32),
            pltpu.VMEM((8, DIM), jnp.float32),
            pltpu.VMEM_SHARED((N_PAD, DIM), jnp.float32),
            pltpu.SemaphoreType.DMA,
            pltpu.SemaphoreType.DMA,
            pltpu.SemaphoreType.DMA,
            pltpu.SemaphoreType.DMA,
            pltpu.SemaphoreType.DMA,
            pltpu.SemaphoreType.DMA,
        ],
        compiler_params=pltpu.CompilerParams(needs_layout_passes=False),
    )
    def body(h2_hbm, ed_hbm, out_hbm,
             eb0, eb1, rows0, rows1, db0, db1, zbuf_v, agg_sh,
             se0, se1, sg0, sg1, ss0, ss1):
        c = lax.axis_index("c")
        s = lax.axis_index("s")
        wid = s * NC + c

        eb = (eb0, eb1)
        rows = (rows0, rows1)
        db = (db0, db1)
        se = (se0, se1)
        sg = (sg0, sg1)
        ss = (ss0, ss1)

        for r in range(8):
            for j in range(DIM // 16):
                zbuf_v[r, pl.ds(j * 16, 16)] = jnp.zeros((16,), jnp.float32)

        def zcopy(k, _):
            pltpu.sync_copy(zbuf_v, agg_sh.at[pl.ds(s * RPT + k * 8, 8)])
            return 0
        lax.fori_loop(0, RPT // 8, zcopy, 0)
        plsc.subcore_barrier()

        cbase = wid * NCH

        def fire_idx(k, p):
            pltpu.async_copy(ed_hbm.at[cbase + k], eb[p], se[p])

        def wait_idx(k, p):
            pltpu.make_async_copy(ed_hbm.at[cbase + k], eb[p], se[p]).wait()

        def fire_gather(p):
            pltpu.async_copy(h2_hbm.at[eb[p].at[0]], rows[p], sg[p])

        def wait_gather(p):
            pltpu.make_async_copy(h2_hbm.at[eb[p].at[0]], rows[p],
                                  sg[p]).wait()

        def fire_scatter(p):
            pltpu.async_copy(rows[p], agg_sh.at[db[p]], ss[p], add=True)

        def wait_scatter(p):
            pltpu.make_async_copy(rows[p], agg_sh.at[db[p]], ss[p]).wait()

        def compute(p):
            for j in range(CH // 16):
                db[p][pl.ds(j * 16, 16)] = eb[p][1, pl.ds(j * 16, 16)]
            two = jnp.full((16,), 2, jnp.int32)

            def rowm(i, _):
                wv = plsc.bitcast(
                    plsc.load_gather(eb[p], [two, lax.broadcast(i, (16,))]),
                    jnp.float32)
                for j in range(DIM // 16):
                    sl = pl.ds(j * 16, 16)
                    rows[p][i, sl] = rows[p][i, sl] * wv
                return 0
            lax.fori_loop(0, CH, rowm, 0)

        fire_idx(0, 0)
        fire_idx(1, 1)
        wait_idx(0, 0)
        fire_gather(0)

        def pair(pr, _):
            for par in range(2):
                k = 2 * pr + par
                p = par
                q = 1 - par
                wait_gather(p)
                compute(p)
                fire_scatter(p)

                @pl.when(k + 2 < NCH)
                def _():
                    fire_idx(k + 2, p)

                @pl.when(k + 1 < NCH)
                def _():
                    wait_idx(k + 1, q)

                    @pl.when(k >= 1)
                    def _():
                        wait_scatter(q)
                    fire_gather(q)
            return 0
        lax.fori_loop(0, NCH // 2, pair, 0)

        wait_scatter(0)
        wait_scatter(1)
        plsc.subcore_barrier()
        pltpu.sync_copy(
            agg_sh.at[pl.ds(s * RPT, RPT)],
            out_hbm.at[c, pl.ds(s * RPT, RPT)],
        )

    return body(h2, edata)


def _comb_body(h2_ref, p0_ref, p1_ref, h3_ref, sum_ref, sq_ref):
    i = pl.program_id(0)
    x = h2_ref[...] + p0_ref[...] + p1_ref[...]
    h3_ref[...] = x

    @pl.when(i == 0)
    def _():
        sum_ref[...] = jnp.zeros_like(sum_ref)
        sq_ref[...] = jnp.zeros_like(sq_ref)

    sum_ref[0:1, :] += jnp.sum(x, axis=0, keepdims=True)
    sq_ref[0:1, :] += jnp.sum(x * x, axis=0, keepdims=True)


def _combine(h2, p0, p1):
    return pl.pallas_call(
        _comb_body,
        grid=(N_BLKS,),
        in_specs=[
            pl.BlockSpec((ROW_BLK, DIM), lambda i: (i, 0)),
            pl.BlockSpec((ROW_BLK, DIM), lambda i: (i, 0)),
            pl.BlockSpec((ROW_BLK, DIM), lambda i: (i, 0)),
        ],
        out_specs=[
            pl.BlockSpec((ROW_BLK, DIM), lambda i: (i, 0)),
            pl.BlockSpec((8, DIM), lambda i: (0, 0)),
            pl.BlockSpec((8, DIM), lambda i: (0, 0)),
        ],
        out_shape=[
            jax.ShapeDtypeStruct((N_NODES, DIM), jnp.float32),
            jax.ShapeDtypeStruct((8, DIM), jnp.float32),
            jax.ShapeDtypeStruct((8, DIM), jnp.float32),
        ],
    )(h2, p0, p1)


def _bn_body(h3_ref, sum_ref, sq_ref, g_ref, be_ref, o_ref):
    n = jnp.float32(N_NODES)
    mean = sum_ref[0:1, :] / n
    var = sq_ref[0:1, :] / n - mean * mean
    inv = lax.rsqrt(var + BN_EPS)
    o_ref[...] = g_ref[...] * (h3_ref[...] - mean) * inv + be_ref[...]


def _bn_apply(h3, ssum, ssq, g2, be2):
    return pl.pallas_call(
        _bn_body,
        grid=(N_BLKS,),
        in_specs=[
            pl.BlockSpec((ROW_BLK, DIM), lambda i: (i, 0)),
            pl.BlockSpec((8, DIM), lambda i: (0, 0)),
            pl.BlockSpec((8, DIM), lambda i: (0, 0)),
            pl.BlockSpec((1, DIM), lambda i: (0, 0)),
            pl.BlockSpec((1, DIM), lambda i: (0, 0)),
        ],
        out_specs=pl.BlockSpec((ROW_BLK, DIM), lambda i: (i, 0)),
        out_shape=jax.ShapeDtypeStruct((N_NODES, DIM), jnp.float32),
    )(h3, ssum, ssq, g2, be2)


def kernel(h, edge_index, edge_weight, W, b, gamma, beta):
    src = edge_index[0].astype(jnp.int32)
    dst = edge_index[1].astype(jnp.int32)
    pad = E_PAD - N_EDGES
    src = jnp.concatenate([src, jnp.zeros((pad,), jnp.int32)])
    dst = jnp.concatenate([dst, jnp.zeros((pad,), jnp.int32)])
    ew = jnp.concatenate([edge_weight.astype(jnp.float32),
                          jnp.zeros((pad,), jnp.float32)])
    nch_tot = E_PAD // CH
    edata = jnp.stack(
        [src.reshape(nch_tot, CH), dst.reshape(nch_tot, CH),
         lax.bitcast_convert_type(ew, jnp.int32).reshape(nch_tot, CH)],
        axis=1)

    h2 = _project(h, W, b.reshape(1, DIM))
    parts = _sc_aggregate(h2, edata)
    h3, ssum, ssq = _combine(h2, parts[0, :N_NODES], parts[1, :N_NODES])
    return _bn_apply(h3, ssum, ssq, gamma.reshape(1, DIM),
                     beta.reshape(1, DIM))
